# baseline (speedup 1.0000x reference)
import jax
import jax.numpy as jnp
from jax.experimental import pallas as pl

NZ = 64
N_ETAS = 8
TN = 512


def _fused_kernel(xq_ref, xs_ref, xt_ref, eta_ref,
                  wb_ref, bb_ref, wsh_ref, wpr_ref, bc_ref, out_ref):
    xq = xq_ref[...]
    xs = xs_ref[...]
    xt = xt_ref[...]
    X = jnp.concatenate([xq, xs, xt], axis=0)
    predcat = jax.lax.dot_general(
        wb_ref[...], X, (((1,), (0,)), ((), ())),
        preferred_element_type=jnp.float32) + bb_ref[...]
    eta = eta_ref[0]
    acc = predcat
    for e in range(N_ETAS):
        r = jax.lax.dot_general(
            wsh_ref[e], X, (((0,), (0,)), ((), ())),
            preferred_element_type=jnp.float32)
        r = r + jax.lax.dot_general(
            wpr_ref[e], predcat, (((0,), (0,)), ((), ())),
            preferred_element_type=jnp.float32)
        r = r + bc_ref[e]
        acc = acc + jnp.where(eta == e, r, 0.0)
    out_ref[...] = acc


def kernel(x_QT, x_SLI, x_SST, eta, W_base_QT, b_base_QT, W_base_SLI,
           b_base_SLI, W_ratio_QT, b_ratio_QT, W_ratio_SLI, b_ratio_SLI):
    nz, h, w = x_QT.shape
    N = h * w
    E, FEAT, _ = W_ratio_QT.shape
    xq = x_QT.reshape(nz, N)
    xs = x_SLI.reshape(nz, N)
    xt = x_SST.reshape(nz, N)
    T = N // TN
    eta3 = eta.reshape(T, 1, TN).astype(jnp.int32)

    perm = jnp.concatenate([jnp.arange(2 * nz, 3 * nz),
                            jnp.arange(3 * nz, 4 * nz),
                            jnp.arange(nz, 2 * nz)])
    wsh = jnp.concatenate([W_ratio_QT[:, perm, :],
                           W_ratio_SLI[:, perm, :]], axis=2)
    z = jnp.zeros((E, nz, nz), jnp.float32)
    top = jnp.concatenate([W_ratio_QT[:, :nz, :], z], axis=2)
    bot = jnp.concatenate([z, W_ratio_SLI[:, :nz, :]], axis=2)
    wpr = jnp.concatenate([top, bot], axis=1)
    bc = jnp.concatenate([b_ratio_QT, b_ratio_SLI],
                         axis=1)[:, :, None]
    wb = jnp.concatenate([W_base_QT, W_base_SLI], axis=0)
    bb = jnp.concatenate([b_base_QT, b_base_SLI])[:, None]

    out = pl.pallas_call(
        _fused_kernel,
        grid=(T,),
        in_specs=[
            pl.BlockSpec((nz, TN), lambda t: (0, t)),
            pl.BlockSpec((nz, TN), lambda t: (0, t)),
            pl.BlockSpec((nz, TN), lambda t: (0, t)),
            pl.BlockSpec((1, 1, TN), lambda t: (t, 0, 0)),
            pl.BlockSpec((2 * nz, 3 * nz), lambda t: (0, 0)),
            pl.BlockSpec((2 * nz, 1), lambda t: (0, 0)),
            pl.BlockSpec((E, 3 * nz, 2 * nz), lambda t: (0, 0, 0)),
            pl.BlockSpec((E, 2 * nz, 2 * nz), lambda t: (0, 0, 0)),
            pl.BlockSpec((E, 2 * nz, 1), lambda t: (0, 0, 0)),
        ],
        out_specs=pl.BlockSpec((2 * nz, TN), lambda t: (0, t)),
        out_shape=jax.ShapeDtypeStruct((2 * nz, N), jnp.float32),
    )(xq, xs, xt, eta3, wb, bb, wsh, wpr, bc)

    return out.reshape(2, nz, h, w)


# speedup vs baseline: 1.0185x; 1.0185x over previous
import jax
import jax.numpy as jnp
from jax.experimental import pallas as pl

NZ = 64
N_ETAS = 8
TN = 512


def _fused_kernel(xq_ref, xs_ref, xt_ref, eta_ref,
                  wb_ref, bb_ref, wsh_ref, wpr_ref, bc_ref, out_ref):
    xq = xq_ref[...]
    xs = xs_ref[...]
    xt = xt_ref[...]
    X = jnp.concatenate([xq, xs, xt], axis=0)
    predcat = jax.lax.dot_general(
        wb_ref[...], X, (((1,), (0,)), ((), ())),
        preferred_element_type=jnp.float32) + bb_ref[...]
    predb = predcat.astype(jnp.bfloat16)
    eta = eta_ref[0]
    acc = predcat
    for e in range(N_ETAS):
        r = jax.lax.dot_general(
            wsh_ref[e], X, (((0,), (0,)), ((), ())),
            preferred_element_type=jnp.float32)
        r = r + jax.lax.dot_general(
            wpr_ref[e], predb, (((0,), (0,)), ((), ())),
            preferred_element_type=jnp.float32)
        r = r + bc_ref[e]
        acc = acc + jnp.where(eta == e, r, 0.0)
    out_ref[...] = acc


def kernel(x_QT, x_SLI, x_SST, eta, W_base_QT, b_base_QT, W_base_SLI,
           b_base_SLI, W_ratio_QT, b_ratio_QT, W_ratio_SLI, b_ratio_SLI):
    nz, h, w = x_QT.shape
    N = h * w
    E, FEAT, _ = W_ratio_QT.shape
    xq = x_QT.reshape(nz, N).astype(jnp.bfloat16)
    xs = x_SLI.reshape(nz, N).astype(jnp.bfloat16)
    xt = x_SST.reshape(nz, N).astype(jnp.bfloat16)
    T = N // TN
    eta3 = eta.reshape(T, 1, TN).astype(jnp.int32)

    perm = jnp.concatenate([jnp.arange(2 * nz, 3 * nz),
                            jnp.arange(3 * nz, 4 * nz),
                            jnp.arange(nz, 2 * nz)])
    wsh = jnp.concatenate([W_ratio_QT[:, perm, :],
                           W_ratio_SLI[:, perm, :]],
                          axis=2).astype(jnp.bfloat16)
    z = jnp.zeros((E, nz, nz), jnp.float32)
    top = jnp.concatenate([W_ratio_QT[:, :nz, :], z], axis=2)
    bot = jnp.concatenate([z, W_ratio_SLI[:, :nz, :]], axis=2)
    wpr = jnp.concatenate([top, bot],
                          axis=1).astype(jnp.bfloat16)
    bc = jnp.concatenate([b_ratio_QT, b_ratio_SLI],
                         axis=1)[:, :, None]
    wb = jnp.concatenate([W_base_QT, W_base_SLI],
                         axis=0).astype(jnp.bfloat16)
    bb = jnp.concatenate([b_base_QT, b_base_SLI])[:, None]

    out = pl.pallas_call(
        _fused_kernel,
        grid=(T,),
        in_specs=[
            pl.BlockSpec((nz, TN), lambda t: (0, t)),
            pl.BlockSpec((nz, TN), lambda t: (0, t)),
            pl.BlockSpec((nz, TN), lambda t: (0, t)),
            pl.BlockSpec((1, 1, TN), lambda t: (t, 0, 0)),
            pl.BlockSpec((2 * nz, 3 * nz), lambda t: (0, 0)),
            pl.BlockSpec((2 * nz, 1), lambda t: (0, 0)),
            pl.BlockSpec((E, 3 * nz, 2 * nz), lambda t: (0, 0, 0)),
            pl.BlockSpec((E, 2 * nz, 2 * nz), lambda t: (0, 0, 0)),
            pl.BlockSpec((E, 2 * nz, 1), lambda t: (0, 0, 0)),
        ],
        out_specs=pl.BlockSpec((2 * nz, TN), lambda t: (0, t)),
        out_shape=jax.ShapeDtypeStruct((2 * nz, N), jnp.float32),
    )(xq, xs, xt, eta3, wb, bb, wsh, wpr, bc)

    return out.reshape(2, nz, h, w)
